# Initial kernel scaffold; baseline (speedup 1.0000x reference)
import jax
import jax.numpy as jnp
from jax.experimental import pallas as pl


def kernel(anchors, prob):
    raise NotImplementedError("write your pallas kernel here")


# speedup vs baseline: 4.0797x; 4.0797x over previous
import functools

import jax
import jax.numpy as jnp
from jax.experimental import pallas as pl

_TOPK = 10
_ENTROPY_WEIGHT = 2.0
_EPS = 1e-08
_BLK = 256


def _softmax_entropy_kernel(p_ref, q_ref, ent_ref):
    p = p_ref[...]
    m = jnp.max(p, axis=1, keepdims=True)
    e = jnp.exp(p - m)
    q = e / jnp.sum(e, axis=1, keepdims=True)
    q_ref[...] = q
    col_mean = jnp.mean(q, axis=0)
    x = jnp.clip(col_mean, _EPS, None)
    ent_ref[...] = jnp.reshape(-jnp.sum(x * jnp.log(x)), (1, 1))


def _knn_loss_kernel(a_blk_ref, a_t_ref, q_ref, acc_ref):
    i = pl.program_id(0)
    feats = jax.lax.dot_general(
        a_blk_ref[...], a_t_ref[...],
        (((1,), (0,)), ((), ())),
        precision=jax.lax.Precision.HIGHEST,
        preferred_element_type=jnp.float32,
    )
    blk, n_rows = feats.shape
    lane = jax.lax.broadcasted_iota(jnp.int32, (blk, n_rows), 1)
    q = q_ref[...]
    qb = q_ref[pl.ds(i * blk, blk), :]
    sim = jnp.zeros((blk, _TOPK), jnp.float32)
    work = feats
    for t in range(_TOPK + 1):
        m = jnp.max(work, axis=1, keepdims=True)
        idx = jnp.min(
            jnp.where(work == m, lane, n_rows), axis=1, keepdims=True
        )
        onehot = lane == idx
        if t > 0:
            g = jax.lax.dot_general(
                onehot.astype(jnp.float32), q,
                (((1,), (0,)), ((), ())),
                precision=jax.lax.Precision.HIGHEST,
                preferred_element_type=jnp.float32,
            )
            u = t - 1
            for v in range(_TOPK):
                w = qb[:, 10 * u + v][:, None]
                sim = sim + w * g[:, 10 * v:10 * v + _TOPK]
        if t < _TOPK:
            work = jnp.where(onehot, -jnp.inf, work)
    log_sim = jnp.clip(jnp.log(sim), -100.0, None)

    @pl.when(i == 0)
    def _init():
        acc_ref[...] = jnp.zeros((1, 1), jnp.float32)

    acc_ref[...] += jnp.reshape(jnp.sum(log_sim), (1, 1))


@functools.partial(jax.jit, static_argnames=())
def kernel(anchors, prob):
    b, c, h, w = anchors.shape
    n_rows = b * h * w
    a = jnp.transpose(anchors, (0, 3, 2, 1)).reshape(n_rows, c)
    p = jnp.transpose(prob, (0, 3, 2, 1)).reshape(n_rows, -1)

    q, ent = pl.pallas_call(
        _softmax_entropy_kernel,
        out_shape=(
            jax.ShapeDtypeStruct((n_rows, p.shape[1]), jnp.float32),
            jax.ShapeDtypeStruct((1, 1), jnp.float32),
        ),
    )(p)

    num_blocks = n_rows // _BLK
    acc = pl.pallas_call(
        _knn_loss_kernel,
        grid=(num_blocks,),
        in_specs=[
            pl.BlockSpec((_BLK, c), lambda i: (i, 0)),
            pl.BlockSpec((c, n_rows), lambda i: (0, 0)),
            pl.BlockSpec((n_rows, p.shape[1]), lambda i: (0, 0)),
        ],
        out_specs=pl.BlockSpec((1, 1), lambda i: (0, 0)),
        out_shape=jax.ShapeDtypeStruct((1, 1), jnp.float32),
    )(a, a.T, q)

    consistency = -acc[0, 0] / (n_rows * _TOPK)
    entropy = ent[0, 0]
    total = consistency - _ENTROPY_WEIGHT * entropy
    return (total, consistency, entropy)


# speedup vs baseline: 12.0933x; 2.9643x over previous
import functools

import jax
import jax.numpy as jnp
from jax.experimental import pallas as pl

_TOPK = 10
_ENTROPY_WEIGHT = 2.0
_EPS = 1e-08
_BLK = 256


def _softmax_entropy_kernel(p_ref, q_ref, ent_ref):
    p = p_ref[...]
    m = jnp.max(p, axis=1, keepdims=True)
    e = jnp.exp(p - m)
    q = e / jnp.sum(e, axis=1, keepdims=True)
    q_ref[...] = q
    col_mean = jnp.mean(q, axis=0)
    x = jnp.clip(col_mean, _EPS, None)
    ent_ref[...] = jnp.reshape(-jnp.sum(x * jnp.log(x)), (1, 1))


def _knn_loss_kernel(a_blk_ref, a_t_ref, q_ref, acc_ref):
    i = pl.program_id(0)
    feats = jax.lax.dot_general(
        a_blk_ref[...], a_t_ref[...],
        (((1,), (0,)), ((), ())),
        precision=jax.lax.Precision.HIGHEST,
        preferred_element_type=jnp.float32,
    )
    blk, n_rows = feats.shape
    lane = jax.lax.broadcasted_iota(jnp.int32, (blk, n_rows), 1)
    q = q_ref[...]
    qb = q_ref[pl.ds(i * blk, blk), :]
    sim = jnp.zeros((blk, _TOPK), jnp.float32)
    work = feats
    for t in range(_TOPK + 1):
        m = jnp.max(work, axis=1, keepdims=True)
        idx = jnp.min(
            jnp.where(work == m, lane, n_rows), axis=1, keepdims=True
        )
        onehot = lane == idx
        if t > 0:
            g = jax.lax.dot_general(
                onehot.astype(jnp.float32), q,
                (((1,), (0,)), ((), ())),
                precision=jax.lax.Precision.DEFAULT,
                preferred_element_type=jnp.float32,
            )
            u = t - 1
            for v in range(_TOPK):
                w = qb[:, 10 * u + v][:, None]
                sim = sim + w * g[:, 10 * v:10 * v + _TOPK]
        if t < _TOPK:
            work = jnp.where(onehot, -jnp.inf, work)
    log_sim = jnp.clip(jnp.log(sim), -100.0, None)

    @pl.when(i == 0)
    def _init():
        acc_ref[...] = jnp.zeros((1, 1), jnp.float32)

    acc_ref[...] += jnp.reshape(jnp.sum(log_sim), (1, 1))


@functools.partial(jax.jit, static_argnames=())
def kernel(anchors, prob):
    b, c, h, w = anchors.shape
    n_rows = b * h * w
    a = jnp.transpose(anchors, (0, 3, 2, 1)).reshape(n_rows, c)
    p = jnp.transpose(prob, (0, 3, 2, 1)).reshape(n_rows, -1)

    q, ent = pl.pallas_call(
        _softmax_entropy_kernel,
        out_shape=(
            jax.ShapeDtypeStruct((n_rows, p.shape[1]), jnp.float32),
            jax.ShapeDtypeStruct((1, 1), jnp.float32),
        ),
    )(p)

    num_blocks = n_rows // _BLK
    acc = pl.pallas_call(
        _knn_loss_kernel,
        grid=(num_blocks,),
        in_specs=[
            pl.BlockSpec((_BLK, c), lambda i: (i, 0)),
            pl.BlockSpec((c, n_rows), lambda i: (0, 0)),
            pl.BlockSpec((n_rows, p.shape[1]), lambda i: (0, 0)),
        ],
        out_specs=pl.BlockSpec((1, 1), lambda i: (0, 0)),
        out_shape=jax.ShapeDtypeStruct((1, 1), jnp.float32),
    )(a, a.T, q)

    consistency = -acc[0, 0] / (n_rows * _TOPK)
    entropy = ent[0, 0]
    total = consistency - _ENTROPY_WEIGHT * entropy
    return (total, consistency, entropy)
